# baseline (speedup 1.0000x reference)
import jax
import jax.numpy as jnp
from jax import lax
from jax.experimental import pallas as pl
from jax.experimental.pallas import tpu as pltpu
from jax.experimental.pallas import tpu_sc as plsc

B = 16384
D = 64
VT = 100000
VC = 1000
NC, NS = 2, 16
NW = NC * NS
FPW = D // NW
CHUNK = 4096
NCHUNK = B // CHUNK
LANES = 16


UNROLL = 8


def _body(pt_hbm, pc_hbm, pidx_hbm, out_hbm,
          row_v, crow_v, pidx_v, outc_v, sem_in, sem_out):
    wid = lax.axis_index("s") * NC + lax.axis_index("c")
    tmask = jnp.int32((1 << 17) - 1)
    ngl = FPW * NCHUNK
    idx_cps = {0: pltpu.async_copy(
        pidx_hbm.at[pl.ds(0, CHUNK)], pidx_v.at[0], sem_in)}
    out_pending = []
    for j in range(FPW):
        f = wid * FPW + j
        row_cp = pltpu.async_copy(pt_hbm.at[f], row_v, sem_in)
        crow_cp = pltpu.async_copy(pc_hbm.at[f], crow_v, sem_in)
        row_cp.wait()
        crow_cp.wait()
        for ch in range(NCHUNK):
            g = j * NCHUNK + ch
            s = g % 2
            so = g % 4
            idx_cps.pop(g).wait()
            if g + 1 < ngl:
                nsl = pl.ds(((g + 1) % NCHUNK) * CHUNK, CHUNK)
                idx_cps[g + 1] = pltpu.async_copy(
                    pidx_hbm.at[nsl], pidx_v.at[1 - s], sem_in)
            while len(out_pending) >= 4:
                out_pending.pop(0).wait()

            @plsc.parallel_loop(0, CHUNK // LANES, step=1, unroll=UNROLL)
            def gloop(i):
                sl = pl.ds(i * LANES, LANES)
                pk = pidx_v[s, sl]
                tv = plsc.load_gather(row_v, [pk & tmask])
                cv = plsc.load_gather(
                    crow_v, [lax.shift_right_logical(pk, 17)])
                outc_v[so, sl] = tv + cv
            out_pending.append(pltpu.async_copy(
                outc_v.at[so], out_hbm.at[f, pl.ds(ch * CHUNK, CHUNK)], sem_out))
    for cp in out_pending:
        cp.wait()


def kernel(team_skill, conf_skill, team, conf):
    pt = team_skill.T
    pc = conf_skill.T
    tidx = team.reshape(-1).astype(jnp.int32)
    cidx = conf.reshape(-1).astype(jnp.int32)
    pidx = tidx | (cidx << 17)
    mesh = plsc.VectorSubcoreMesh(
        core_axis_name="c", subcore_axis_name="s",
        num_cores=NC, num_subcores=NS)
    f = pl.kernel(
        _body,
        out_type=jax.ShapeDtypeStruct((D, B), jnp.float32),
        mesh=mesh,
        scratch_types=[
            pltpu.VMEM((VT,), jnp.float32),
            pltpu.VMEM((VC,), jnp.float32),
            pltpu.VMEM((2, CHUNK), jnp.int32),
            pltpu.VMEM((4, CHUNK), jnp.float32),
            pltpu.SemaphoreType.DMA,
            pltpu.SemaphoreType.DMA,
        ],
        compiler_params=pltpu.CompilerParams(
            use_tc_tiling_on_sc=True, needs_layout_passes=False),
    )
    out_t = f(pt, pc, pidx)
    return out_t.T
